# baseline (speedup 1.0000x reference)
import functools
import math

import jax
import jax.numpy as jnp
import numpy as np
from jax.experimental import pallas as pl
from jax.experimental.pallas import tpu as pltpu

N_HEADS = 16
D_MODEL = 1024
D_HEAD = D_MODEL // N_HEADS
TOP_K = 64
CONTEXT_LEN = 2048
NEG_INF = -1e30


def _rope_tables_full(T, d_head):
    position = jnp.arange(T, dtype=jnp.float32)[:, None]
    div_term = 10000.0 ** (jnp.arange(0, d_head, 2, dtype=jnp.float32) / d_head)
    div_term = jnp.repeat(div_term, 2)
    cos = jnp.cos(position / div_term)
    sin = jnp.sin(position / div_term)
    return cos, sin


def _pair_swap_matrix(d_head):
    P = np.zeros((d_head, d_head), dtype=np.float32)
    for i in range(d_head // 2):
        P[2 * i + 1, 2 * i] = -1.0
        P[2 * i, 2 * i + 1] = 1.0
    return jnp.asarray(P)


def _encode(x):
    b = jax.lax.bitcast_convert_type(x, jnp.uint32)
    sign = jnp.uint32(0x80000000)
    return jnp.where(b >= sign, ~b, b | sign)


def _sum_keys(x, W, n):
    return jnp.sum(jnp.sum(x.reshape(W // 8, 8, n), axis=0), axis=0,
                   keepdims=True)


def _attn_kernel(cos_ref, sin_ref, perm_ref, q_ref, k_ref, v_ref, o_ref,
                 kr_ref, l_ref, u_ref, *, bq, T, top_k, nv):
    qi = pl.program_id(1)
    scale = 1.0 / math.sqrt(D_HEAD)
    hi = jax.lax.Precision.HIGHEST

    P = perm_ref[...]

    @pl.when(qi == 0)
    def _():
        kh = k_ref[0]
        kr_ref[...] = kh * cos_ref[...] + jax.lax.dot(
            kh, P, preferred_element_type=jnp.float32, precision=hi
        ) * sin_ref[...]

    qh = q_ref[0]
    qpos = qi * bq
    cq = cos_ref[pl.ds(qpos, bq), :]
    sq = sin_ref[pl.ds(qpos, bq), :]
    qr = qh * cq + jax.lax.dot(
        qh, P, preferred_element_type=jnp.float32, precision=hi) * sq

    for v in range(nv):
        @pl.when(qi == v)
        def _(v=v):
            W = (v + 1) * bq
            kr = kr_ref[0:W, :]
            lg = jax.lax.dot_general(
                kr, qr, (((1,), (1,)), ((), ())),
                preferred_element_type=jnp.float32) * scale
            kidx = jax.lax.broadcasted_iota(jnp.int32, (W, bq), 0)
            qidx = qpos + jax.lax.broadcasted_iota(jnp.int32, (W, bq), 1)
            lg = jnp.where(kidx <= qidx, lg, NEG_INF)
            l_ref[0:W] = lg
            u = _encode(lg)
            u_ref[0:W] = u
            m = jnp.max(jnp.max(lg.reshape(W // 8, 8, bq), axis=0), axis=0,
                        keepdims=True)

            t = jnp.zeros((1, bq), jnp.uint32)
            for i in range(31, -1, -1):
                cand = t | jnp.uint32(1 << i)
                mk = (u_ref[0:W] >= cand).astype(jnp.float32)
                cnt = _sum_keys(mk, W, bq)
                t = jnp.where(cnt >= float(top_k), cand, t)

            e = jnp.exp(l_ref[0:W] - m)
            w = jnp.where(u_ref[0:W] >= t, e, 0.0)
            z = _sum_keys(e, W, bq)
            den = _sum_keys(w, W, bq)
            ov = jax.lax.dot_general(
                v_ref[0, :, 0:W], w, (((1,), (0,)), ((), ())),
                preferred_element_type=jnp.float32,
                precision=hi)
            o_ref[0] = ov / (den + 1e-9 * z)


def kernel(q, k, v):
    b, T, d_model = q.shape
    H, d_head = N_HEADS, D_HEAD
    assert b == 1 and d_model == D_MODEL

    qh = q.reshape(T, H, d_head).transpose(1, 0, 2)
    kh = k.reshape(T, H, d_head).transpose(1, 0, 2)
    vT = v.reshape(T, H, d_head).transpose(1, 2, 0)

    cos, sin = _rope_tables_full(CONTEXT_LEN, d_head)
    cos = cos[:T]
    sin = sin[:T]
    P = _pair_swap_matrix(d_head)

    bq = min(512, T)
    nv = T // bq
    grid = (H, nv)

    out = pl.pallas_call(
        functools.partial(_attn_kernel, bq=bq, T=T, top_k=TOP_K, nv=nv),
        grid=grid,
        in_specs=[
            pl.BlockSpec((T, d_head), lambda h, i: (0, 0)),
            pl.BlockSpec((T, d_head), lambda h, i: (0, 0)),
            pl.BlockSpec((d_head, d_head), lambda h, i: (0, 0)),
            pl.BlockSpec((1, bq, d_head), lambda h, i: (h, i, 0)),
            pl.BlockSpec((1, T, d_head), lambda h, i: (h, 0, 0)),
            pl.BlockSpec((1, d_head, T), lambda h, i: (h, 0, 0)),
        ],
        out_specs=pl.BlockSpec((1, d_head, bq), lambda h, i: (h, 0, i)),
        out_shape=jax.ShapeDtypeStruct((H, d_head, T), jnp.float32),
        scratch_shapes=[
            pltpu.VMEM((T, d_head), jnp.float32),
            pltpu.VMEM((T, bq), jnp.float32),
            pltpu.VMEM((T, bq), jnp.uint32),
        ],
        compiler_params=pltpu.CompilerParams(
            dimension_semantics=("arbitrary", "arbitrary")),
    )(cos, sin, P, qh, kh, vT)

    return out.transpose(2, 0, 1).reshape(1, T, d_model)
